# Initial kernel scaffold; baseline (speedup 1.0000x reference)
import jax
import jax.numpy as jnp
from jax.experimental import pallas as pl


def kernel(v0, edge_index):
    raise NotImplementedError("write your pallas kernel here")


# speedup vs baseline: 57.4005x; 57.4005x over previous
import functools

import jax
import jax.numpy as jnp
from jax import lax
from jax.experimental import pallas as pl
from jax.experimental.pallas import tpu as pltpu
from jax.experimental.pallas import tpu_sc as plsc

N = 100000
D = 8
NC = 2
NS = 16
NW = NC * NS
ROWS_PER_TILE = 6272
NP = NS * ROWS_PER_TILE
SUB = 896
NSUB = ROWS_PER_TILE // SUB
GRP = 128
SGRP = SUB // GRP
E = 3_200_000
CH_E = 1024
CH = CH_E // GRP
EW = 100352
NCHUNK = EW // CH_E
EPAD = EW * NW
ER = EPAD // GRP
ERW = EW // GRP

_mesh = plsc.VectorSubcoreMesh(core_axis_name="c", subcore_axis_name="s")


def _combine_into_shared(parts, buf_a, buf_b, id_ref, shared_v, base):
    lanes = lax.iota(jnp.int32, 16)

    def body(i, carry):
        r0 = base + i * SUB
        pltpu.sync_copy(parts.at[0, pl.ds(r0, SUB)], buf_a)
        pltpu.sync_copy(buf_a, shared_v.at[pl.ds(r0, SUB)])
        pltpu.sync_copy(parts.at[1, pl.ds(r0, SUB)], buf_b)
        for j in range(SGRP):
            for k in range(GRP // 16):
                id_ref[j, pl.ds(k * 16, 16)] = r0 + j * GRP + k * 16 + lanes
        for j in range(SGRP):
            pltpu.sync_copy(
                buf_b.at[pl.ds(j * GRP, GRP)],
                shared_v.at[id_ref.at[j]],
                add=True,
            )
        return carry

    lax.fori_loop(0, NSUB, body, 0)


def _zero_shared_out(zeros_hbm, buf, shared_out, base):
    pltpu.sync_copy(zeros_hbm, buf)
    for i in range(NSUB):
        pltpu.sync_copy(buf, shared_out.at[pl.ds(base + i * SUB, SUB)])


def _edge_phase(src_hbm, dst_hbm, sbuf, dbuf, rows, gsem, shared_v, shared_out, w):
    wr0 = w * ERW

    def chunk(g, carry):
        row0 = wr0 + g * CH
        pltpu.sync_copy(src_hbm.at[pl.ds(row0, CH)], sbuf)
        pltpu.sync_copy(dst_hbm.at[pl.ds(row0, CH)], dbuf)
        descs = [
            pltpu.async_copy(shared_v.at[sbuf.at[j]], rows.at[j], gsem)
            for j in range(CH)
        ]
        for d in descs:
            d.wait()
        for j in range(CH):
            pltpu.sync_copy(rows.at[j], shared_out.at[dbuf.at[j]], add=True)
        return carry

    lax.fori_loop(0, NCHUNK, chunk, 0)


def _writeout_parts(parts_out, buf, shared_out, base, c):
    for i in range(NSUB):
        r0 = base + i * SUB
        pltpu.sync_copy(shared_out.at[pl.ds(r0, SUB)], buf)
        pltpu.sync_copy(buf, parts_out.at[c, pl.ds(r0, SUB)])


@functools.partial(
    pl.kernel,
    out_type=jax.ShapeDtypeStruct((NC, NP, D), jnp.float32),
    mesh=_mesh,
    compiler_params=pltpu.CompilerParams(use_tc_tiling_on_sc=False),
    scratch_types=[
        pltpu.VMEM_SHARED((NP, D), jnp.float32),
        pltpu.VMEM_SHARED((NP, D), jnp.float32),
        pltpu.VMEM((SUB, D), jnp.float32),
        pltpu.VMEM((CH, GRP), jnp.int32),
        pltpu.VMEM((CH, GRP), jnp.int32),
        pltpu.VMEM((CH, GRP, D), jnp.float32),
        pltpu.SemaphoreType.DMA,
    ],
)
def _step_first(v_hbm, src_hbm, dst_hbm, zeros_hbm, parts_out,
                shared_v, shared_out, buf_a, sbuf, dbuf, rows, gsem):
    c = lax.axis_index("c")
    s = lax.axis_index("s")
    base = s * ROWS_PER_TILE
    for i in range(NSUB):
        r0 = base + i * SUB
        pltpu.sync_copy(v_hbm.at[pl.ds(r0, SUB)], buf_a)
        pltpu.sync_copy(buf_a, shared_v.at[pl.ds(r0, SUB)])
    _zero_shared_out(zeros_hbm, buf_a, shared_out, base)
    plsc.subcore_barrier()
    _edge_phase(src_hbm, dst_hbm, sbuf, dbuf, rows, gsem, shared_v, shared_out,
                c * NS + s)
    plsc.subcore_barrier()
    _writeout_parts(parts_out, buf_a, shared_out, base, c)


@functools.partial(
    pl.kernel,
    out_type=jax.ShapeDtypeStruct((NC, NP, D), jnp.float32),
    mesh=_mesh,
    compiler_params=pltpu.CompilerParams(use_tc_tiling_on_sc=False),
    scratch_types=[
        pltpu.VMEM_SHARED((NP, D), jnp.float32),
        pltpu.VMEM_SHARED((NP, D), jnp.float32),
        pltpu.VMEM((SUB, D), jnp.float32),
        pltpu.VMEM((SUB, D), jnp.float32),
        pltpu.VMEM((SGRP, GRP), jnp.int32),
        pltpu.VMEM((CH, GRP), jnp.int32),
        pltpu.VMEM((CH, GRP), jnp.int32),
        pltpu.VMEM((CH, GRP, D), jnp.float32),
        pltpu.SemaphoreType.DMA,
    ],
)
def _step_mid(parts_in, src_hbm, dst_hbm, zeros_hbm, parts_out,
              shared_v, shared_out, buf_a, buf_b, id_ref, sbuf, dbuf, rows,
              gsem):
    c = lax.axis_index("c")
    s = lax.axis_index("s")
    base = s * ROWS_PER_TILE
    _combine_into_shared(parts_in, buf_a, buf_b, id_ref, shared_v, base)
    _zero_shared_out(zeros_hbm, buf_a, shared_out, base)
    plsc.subcore_barrier()
    _edge_phase(src_hbm, dst_hbm, sbuf, dbuf, rows, gsem, shared_v, shared_out,
                c * NS + s)
    plsc.subcore_barrier()
    _writeout_parts(parts_out, buf_a, shared_out, base, c)


@functools.partial(
    pl.kernel,
    out_type=jax.ShapeDtypeStruct((NP, D), jnp.float32),
    mesh=_mesh,
    compiler_params=pltpu.CompilerParams(use_tc_tiling_on_sc=False),
    scratch_types=[
        pltpu.VMEM_SHARED((NP, D), jnp.float32),
        pltpu.VMEM((SUB, D), jnp.float32),
        pltpu.VMEM((SUB, D), jnp.float32),
        pltpu.VMEM((SGRP, GRP), jnp.int32),
        pltpu.VMEM((SUB // 2, D), jnp.float32),
    ],
)
def _step_last(parts_in, v_out, shared_v, buf_a, buf_b, id_ref, wbuf):
    c = lax.axis_index("c")
    s = lax.axis_index("s")
    base = s * ROWS_PER_TILE
    _combine_into_shared(parts_in, buf_a, buf_b, id_ref, shared_v, base)
    half = ROWS_PER_TILE // NC
    hw = SUB // 2
    h0 = base + c * half
    for i in range(half // hw):
        r0 = h0 + i * hw
        pltpu.sync_copy(shared_v.at[pl.ds(r0, hw)], wbuf)
        pltpu.sync_copy(wbuf, v_out.at[pl.ds(r0, hw)])


def kernel(v0, edge_index):
    dst = edge_index[0].astype(jnp.int32)
    src = edge_index[1].astype(jnp.int32)
    src = jnp.concatenate([src, jnp.zeros((EPAD - E,), jnp.int32)])
    dst = jnp.concatenate([dst, jnp.full((EPAD - E,), N, jnp.int32)])
    src2d = src.reshape(ER, GRP)
    dst2d = dst.reshape(ER, GRP)
    v_pad = jnp.pad(v0, ((0, NP - N), (0, 0)))
    zeros = jnp.zeros((SUB, D), jnp.float32)

    parts = _step_first(v_pad, src2d, dst2d, zeros)
    parts = _step_mid(parts, src2d, dst2d, zeros)
    parts = _step_mid(parts, src2d, dst2d, zeros)
    v = _step_last(parts)
    return v[:N]


# speedup vs baseline: 75.9114x; 1.3225x over previous
import functools

import jax
import jax.numpy as jnp
from jax import lax
from jax.experimental import pallas as pl
from jax.experimental.pallas import tpu as pltpu
from jax.experimental.pallas import tpu_sc as plsc

N = 100000
D = 8
NC = 2
NS = 16
NW = NC * NS
ROWS_PER_TILE = 6272
NP = NS * ROWS_PER_TILE
SUB = 128
NSUB = ROWS_PER_TILE // SUB
GRP = 128
E = 3_200_000
CH = 7
CH_E = CH * GRP
NCHUNK = 112
EW = CH_E * NCHUNK
EPAD = EW * NW
ER = EPAD // GRP
ERW = EW // GRP
NSLOT = 3

_mesh = plsc.VectorSubcoreMesh(core_axis_name="c", subcore_axis_name="s")


def _combine_into_shared(parts, buf_a, buf_b, id_ref, shared_v, base):
    lanes = lax.iota(jnp.int32, 16)

    def body(i, carry):
        r0 = base + i * SUB
        pltpu.sync_copy(parts.at[0, pl.ds(r0, SUB)], buf_a)
        pltpu.sync_copy(buf_a, shared_v.at[pl.ds(r0, SUB)])
        pltpu.sync_copy(parts.at[1, pl.ds(r0, SUB)], buf_b)
        for k in range(GRP // 16):
            id_ref[0, pl.ds(k * 16, 16)] = r0 + k * 16 + lanes
        pltpu.sync_copy(buf_b, shared_v.at[id_ref.at[0]], add=True)
        return carry

    lax.fori_loop(0, NSUB, body, 0)


def _zero_shared_out(zeros_hbm, buf, shared_out, base):
    pltpu.sync_copy(zeros_hbm, buf)
    for i in range(NSUB):
        pltpu.sync_copy(buf, shared_out.at[pl.ds(base + i * SUB, SUB)])


def _edge_phase(src_hbm, dst_hbm, sbuf, dbuf, rows, isem, gsem, ssem,
                shared_v, shared_out, w):
    wr0 = w * ERW

    def idx_copies(slot, row0):
        return (
            pltpu.make_async_copy(src_hbm.at[pl.ds(row0, CH)], sbuf.at[slot], isem),
            pltpu.make_async_copy(dst_hbm.at[pl.ds(row0, CH)], dbuf.at[slot], isem),
        )

    def scatter_copies(slot):
        return [
            pltpu.make_async_copy(
                rows.at[slot, j], shared_out.at[dbuf.at[slot, j]], ssem)
            for j in range(CH)
        ]

    for d in idx_copies(0, wr0):
        d.start()

    def chunk(g, carry):
        c = lax.rem(g, NSLOT)
        c1 = lax.rem(g + 1, NSLOT)
        row0 = wr0 + g * CH
        for d in idx_copies(c, row0):
            d.wait()
        @pl.when(g >= 2)
        def _():
            for d in scatter_copies(c1):
                d.wait()
        row_pf = jnp.where(g + 1 < NCHUNK, row0 + CH, wr0)
        for d in idx_copies(c1, row_pf):
            d.start()
        gds = [
            pltpu.make_async_copy(shared_v.at[sbuf.at[c, j]], rows.at[c, j], gsem)
            for j in range(CH)
        ]
        for d in gds:
            d.start()
        for d in gds:
            d.wait()
        for d in scatter_copies(c):
            d.start(add=True)
        return carry

    lax.fori_loop(0, NCHUNK, chunk, 0)

    for g in (NCHUNK - 2, NCHUNK - 1):
        for d in scatter_copies(g % NSLOT):
            d.wait()
    for d in idx_copies(NCHUNK % NSLOT, wr0):
        d.wait()


def _writeout_parts(parts_out, buf, shared_out, base, c):
    for i in range(NSUB):
        r0 = base + i * SUB
        pltpu.sync_copy(shared_out.at[pl.ds(r0, SUB)], buf)
        pltpu.sync_copy(buf, parts_out.at[c, pl.ds(r0, SUB)])


_EDGE_SCRATCH = [
    pltpu.VMEM((NSLOT, CH, GRP), jnp.int32),
    pltpu.VMEM((NSLOT, CH, GRP), jnp.int32),
    pltpu.VMEM((NSLOT, CH, GRP, D), jnp.float32),
    pltpu.SemaphoreType.DMA,
    pltpu.SemaphoreType.DMA,
    pltpu.SemaphoreType.DMA,
]


@functools.partial(
    pl.kernel,
    out_type=jax.ShapeDtypeStruct((NC, NP, D), jnp.float32),
    mesh=_mesh,
    compiler_params=pltpu.CompilerParams(use_tc_tiling_on_sc=False),
    scratch_types=[
        pltpu.VMEM_SHARED((NP, D), jnp.float32),
        pltpu.VMEM_SHARED((NP, D), jnp.float32),
        pltpu.VMEM((SUB, D), jnp.float32),
    ] + _EDGE_SCRATCH,
)
def _step_first(v_hbm, src_hbm, dst_hbm, zeros_hbm, parts_out,
                shared_v, shared_out, buf_a, sbuf, dbuf, rows,
                isem, gsem, ssem):
    c = lax.axis_index("c")
    s = lax.axis_index("s")
    base = s * ROWS_PER_TILE
    for i in range(NSUB):
        r0 = base + i * SUB
        pltpu.sync_copy(v_hbm.at[pl.ds(r0, SUB)], buf_a)
        pltpu.sync_copy(buf_a, shared_v.at[pl.ds(r0, SUB)])
    _zero_shared_out(zeros_hbm, buf_a, shared_out, base)
    plsc.subcore_barrier()
    _edge_phase(src_hbm, dst_hbm, sbuf, dbuf, rows, isem, gsem, ssem,
                shared_v, shared_out, c * NS + s)
    plsc.subcore_barrier()
    _writeout_parts(parts_out, buf_a, shared_out, base, c)


@functools.partial(
    pl.kernel,
    out_type=jax.ShapeDtypeStruct((NC, NP, D), jnp.float32),
    mesh=_mesh,
    compiler_params=pltpu.CompilerParams(use_tc_tiling_on_sc=False),
    scratch_types=[
        pltpu.VMEM_SHARED((NP, D), jnp.float32),
        pltpu.VMEM_SHARED((NP, D), jnp.float32),
        pltpu.VMEM((SUB, D), jnp.float32),
        pltpu.VMEM((SUB, D), jnp.float32),
        pltpu.VMEM((1, GRP), jnp.int32),
    ] + _EDGE_SCRATCH,
)
def _step_mid(parts_in, src_hbm, dst_hbm, zeros_hbm, parts_out,
              shared_v, shared_out, buf_a, buf_b, id_ref, sbuf, dbuf, rows,
              isem, gsem, ssem):
    c = lax.axis_index("c")
    s = lax.axis_index("s")
    base = s * ROWS_PER_TILE
    _combine_into_shared(parts_in, buf_a, buf_b, id_ref, shared_v, base)
    _zero_shared_out(zeros_hbm, buf_a, shared_out, base)
    plsc.subcore_barrier()
    _edge_phase(src_hbm, dst_hbm, sbuf, dbuf, rows, isem, gsem, ssem,
                shared_v, shared_out, c * NS + s)
    plsc.subcore_barrier()
    _writeout_parts(parts_out, buf_a, shared_out, base, c)


@functools.partial(
    pl.kernel,
    out_type=jax.ShapeDtypeStruct((NP, D), jnp.float32),
    mesh=_mesh,
    compiler_params=pltpu.CompilerParams(use_tc_tiling_on_sc=False),
    scratch_types=[
        pltpu.VMEM_SHARED((NP, D), jnp.float32),
        pltpu.VMEM((SUB, D), jnp.float32),
        pltpu.VMEM((SUB, D), jnp.float32),
        pltpu.VMEM((1, GRP), jnp.int32),
        pltpu.VMEM((224, D), jnp.float32),
    ],
)
def _step_last(parts_in, v_out, shared_v, buf_a, buf_b, id_ref, wbuf):
    c = lax.axis_index("c")
    s = lax.axis_index("s")
    base = s * ROWS_PER_TILE
    _combine_into_shared(parts_in, buf_a, buf_b, id_ref, shared_v, base)
    half = ROWS_PER_TILE // NC
    h0 = base + c * half
    for i in range(half // 224):
        r0 = h0 + i * 224
        pltpu.sync_copy(shared_v.at[pl.ds(r0, 224)], wbuf)
        pltpu.sync_copy(wbuf, v_out.at[pl.ds(r0, 224)])


def kernel(v0, edge_index):
    dst = edge_index[0].astype(jnp.int32)
    src = edge_index[1].astype(jnp.int32)
    src = jnp.concatenate([src, jnp.zeros((EPAD - E,), jnp.int32)])
    dst = jnp.concatenate([dst, jnp.full((EPAD - E,), N, jnp.int32)])
    src2d = src.reshape(ER, GRP)
    dst2d = dst.reshape(ER, GRP)
    v_pad = jnp.pad(v0, ((0, NP - N), (0, 0)))
    zeros = jnp.zeros((SUB, D), jnp.float32)

    parts = _step_first(v_pad, src2d, dst2d, zeros)
    parts = _step_mid(parts, src2d, dst2d, zeros)
    parts = _step_mid(parts, src2d, dst2d, zeros)
    v = _step_last(parts)
    return v[:N]


# speedup vs baseline: 77.9371x; 1.0267x over previous
import functools

import jax
import jax.numpy as jnp
from jax import lax
from jax.experimental import pallas as pl
from jax.experimental.pallas import tpu as pltpu
from jax.experimental.pallas import tpu_sc as plsc

N = 100000
D = 8
NC = 2
NS = 16
NW = NC * NS
ROWS_PER_TILE = 6272
NP = NS * ROWS_PER_TILE
SUB = 128
NSUB = ROWS_PER_TILE // SUB
E = 3_200_000
CH_E = 512
NCHUNK = 196
EW = CH_E * NCHUNK
EPAD = EW * NW
ER = EPAD // CH_E
NSLOT = 3

_mesh = plsc.VectorSubcoreMesh(core_axis_name="c", subcore_axis_name="s")


def _build_identity(id_ref, base):
    lanes = lax.iota(jnp.int32, 16)

    def body(i, carry):
        for k in range(SUB // 16):
            id_ref[i, pl.ds(k * 16, 16)] = base + i * SUB + k * 16 + lanes
        return carry

    lax.fori_loop(0, NSUB, body, 0)


def _combine_into_shared(parts, buf_a, buf_b, id_ref, shared_v, base,
                         csem, lsem, ssem):
    def loads(slot, i):
        r0 = base + i * SUB
        return (
            pltpu.make_async_copy(parts.at[0, pl.ds(r0, SUB)], buf_a.at[slot], csem),
            pltpu.make_async_copy(parts.at[1, pl.ds(r0, SUB)], buf_b.at[slot], csem),
        )

    def scat(slot, i):
        return pltpu.make_async_copy(buf_b.at[slot], shared_v.at[id_ref.at[i]], ssem)

    for d in loads(0, 0) + loads(1, 1):
        d.start()

    def body(i, carry):
        cur = lax.rem(i, NSLOT)
        pf = lax.rem(i + 2, NSLOT)
        for d in loads(cur, i):
            d.wait()
        @pl.when(i >= 1)
        def _():
            scat(pf, i - 1).wait()
        for d in loads(pf, lax.rem(i + 2, NSUB)):
            d.start()
        ld = pltpu.make_async_copy(
            buf_a.at[cur], shared_v.at[pl.ds(base + i * SUB, SUB)], lsem)
        ld.start()
        ld.wait()
        scat(cur, i).start(add=True)
        return carry

    lax.fori_loop(0, NSUB, body, 0)

    scat((NSUB - 1) % NSLOT, 0).wait()
    for slot in (NSUB % NSLOT, (NSUB + 1) % NSLOT):
        for d in loads(slot, 0):
            d.wait()


def _zero_shared_out(zeros_hbm, buf, shared_out, base):
    pltpu.sync_copy(zeros_hbm, buf)
    for i in range(NSUB):
        pltpu.sync_copy(buf, shared_out.at[pl.ds(base + i * SUB, SUB)])


def _edge_phase(edges_hbm, ibuf, rows, isem, gsem, ssem,
                shared_v, shared_out, w):
    wr0 = w * NCHUNK

    def idx_copy(slot, row):
        return pltpu.make_async_copy(edges_hbm.at[row], ibuf.at[slot], isem)

    def scatter_copy(slot):
        return pltpu.make_async_copy(
            rows.at[slot], shared_out.at[ibuf.at[slot, 1]], ssem)

    idx_copy(0, wr0).start()

    def chunk(g, carry):
        c = lax.rem(g, NSLOT)
        c1 = lax.rem(g + 1, NSLOT)
        idx_copy(c, wr0 + g).wait()
        @pl.when(g >= 2)
        def _():
            scatter_copy(c1).wait()
        row_pf = jnp.where(g + 1 < NCHUNK, wr0 + g + 1, wr0)
        idx_copy(c1, row_pf).start()
        gd = pltpu.make_async_copy(shared_v.at[ibuf.at[c, 0]], rows.at[c], gsem)
        gd.start()
        gd.wait()
        scatter_copy(c).start(add=True)
        return carry

    lax.fori_loop(0, NCHUNK, chunk, 0)

    for g in (NCHUNK - 2, NCHUNK - 1):
        scatter_copy(g % NSLOT).wait()
    idx_copy(NCHUNK % NSLOT, wr0).wait()


def _writeout_parts(parts_out, buf, shared_out, base, c):
    for i in range(NSUB):
        r0 = base + i * SUB
        pltpu.sync_copy(shared_out.at[pl.ds(r0, SUB)], buf)
        pltpu.sync_copy(buf, parts_out.at[c, pl.ds(r0, SUB)])


_EDGE_SCRATCH = [
    pltpu.VMEM((NSLOT, 2, CH_E), jnp.int32),
    pltpu.VMEM((NSLOT, CH_E, D), jnp.float32),
    pltpu.SemaphoreType.DMA,
    pltpu.SemaphoreType.DMA,
    pltpu.SemaphoreType.DMA,
]


@functools.partial(
    pl.kernel,
    out_type=jax.ShapeDtypeStruct((NC, NP, D), jnp.float32),
    mesh=_mesh,
    compiler_params=pltpu.CompilerParams(use_tc_tiling_on_sc=False),
    scratch_types=[
        pltpu.VMEM_SHARED((NP, D), jnp.float32),
        pltpu.VMEM_SHARED((NP, D), jnp.float32),
        pltpu.VMEM((NSLOT, SUB, D), jnp.float32),
    ] + _EDGE_SCRATCH,
)
def _step_first(v_hbm, edges_hbm, zeros_hbm, parts_out,
                shared_v, shared_out, buf_a, ibuf, rows, isem, gsem, ssem):
    c = lax.axis_index("c")
    s = lax.axis_index("s")
    base = s * ROWS_PER_TILE
    for i in range(NSUB):
        r0 = base + i * SUB
        pltpu.sync_copy(v_hbm.at[pl.ds(r0, SUB)], buf_a.at[0])
        pltpu.sync_copy(buf_a.at[0], shared_v.at[pl.ds(r0, SUB)])
    _zero_shared_out(zeros_hbm, buf_a.at[0], shared_out, base)
    plsc.subcore_barrier()
    _edge_phase(edges_hbm, ibuf, rows, isem, gsem, ssem,
                shared_v, shared_out, c * NS + s)
    plsc.subcore_barrier()
    _writeout_parts(parts_out, buf_a.at[0], shared_out, base, c)


@functools.partial(
    pl.kernel,
    out_type=jax.ShapeDtypeStruct((NC, NP, D), jnp.float32),
    mesh=_mesh,
    compiler_params=pltpu.CompilerParams(use_tc_tiling_on_sc=False),
    scratch_types=[
        pltpu.VMEM_SHARED((NP, D), jnp.float32),
        pltpu.VMEM_SHARED((NP, D), jnp.float32),
        pltpu.VMEM((NSLOT, SUB, D), jnp.float32),
        pltpu.VMEM((NSLOT, SUB, D), jnp.float32),
        pltpu.VMEM((NSUB, SUB), jnp.int32),
    ] + _EDGE_SCRATCH,
)
def _step_mid(parts_in, edges_hbm, zeros_hbm, parts_out,
              shared_v, shared_out, buf_a, buf_b, id_ref, ibuf, rows,
              isem, gsem, ssem):
    c = lax.axis_index("c")
    s = lax.axis_index("s")
    base = s * ROWS_PER_TILE
    _build_identity(id_ref, base)
    _combine_into_shared(parts_in, buf_a, buf_b, id_ref, shared_v, base,
                         isem, gsem, ssem)
    _zero_shared_out(zeros_hbm, buf_a.at[0], shared_out, base)
    plsc.subcore_barrier()
    _edge_phase(edges_hbm, ibuf, rows, isem, gsem, ssem,
                shared_v, shared_out, c * NS + s)
    plsc.subcore_barrier()
    _writeout_parts(parts_out, buf_a.at[0], shared_out, base, c)


@functools.partial(
    pl.kernel,
    out_type=jax.ShapeDtypeStruct((NP, D), jnp.float32),
    mesh=_mesh,
    compiler_params=pltpu.CompilerParams(use_tc_tiling_on_sc=False),
    scratch_types=[
        pltpu.VMEM_SHARED((NP, D), jnp.float32),
        pltpu.VMEM((NSLOT, SUB, D), jnp.float32),
        pltpu.VMEM((NSLOT, SUB, D), jnp.float32),
        pltpu.VMEM((NSUB, SUB), jnp.int32),
        pltpu.VMEM((224, D), jnp.float32),
        pltpu.SemaphoreType.DMA,
        pltpu.SemaphoreType.DMA,
        pltpu.SemaphoreType.DMA,
    ],
)
def _step_last(parts_in, v_out, shared_v, buf_a, buf_b, id_ref, wbuf,
               csem, lsem, ssem):
    c = lax.axis_index("c")
    s = lax.axis_index("s")
    base = s * ROWS_PER_TILE
    _build_identity(id_ref, base)
    _combine_into_shared(parts_in, buf_a, buf_b, id_ref, shared_v, base,
                         csem, lsem, ssem)
    half = ROWS_PER_TILE // NC
    h0 = base + c * half
    for i in range(half // 224):
        r0 = h0 + i * 224
        pltpu.sync_copy(shared_v.at[pl.ds(r0, 224)], wbuf)
        pltpu.sync_copy(wbuf, v_out.at[pl.ds(r0, 224)])


def kernel(v0, edge_index):
    dst = edge_index[0].astype(jnp.int32)
    src = edge_index[1].astype(jnp.int32)
    src = jnp.concatenate([src, jnp.zeros((EPAD - E,), jnp.int32)])
    dst = jnp.concatenate([dst, jnp.full((EPAD - E,), N, jnp.int32)])
    edges = jnp.stack([src.reshape(ER, CH_E), dst.reshape(ER, CH_E)], axis=1)
    v_pad = jnp.pad(v0, ((0, NP - N), (0, 0)))
    zeros = jnp.zeros((SUB, D), jnp.float32)

    parts = _step_first(v_pad, edges, zeros)
    parts = _step_mid(parts, edges, zeros)
    parts = _step_mid(parts, edges, zeros)
    v = _step_last(parts)
    return v[:N]


# speedup vs baseline: 91.1651x; 1.1697x over previous
import functools

import jax
import jax.numpy as jnp
from jax import lax
from jax.experimental import pallas as pl
from jax.experimental.pallas import tpu as pltpu
from jax.experimental.pallas import tpu_sc as plsc

N = 100000
D = 8
NC = 2
NS = 16
NW = NC * NS
ROWS_PER_TILE = 6272
NP = NS * ROWS_PER_TILE
SUB = 128
NSUB = ROWS_PER_TILE // SUB
E = 3_200_000
CH_E = 512
NCK = E // CH_E
NBIG = NCK - 195 * NW
NSLOT = 3

_mesh = plsc.VectorSubcoreMesh(core_axis_name="c", subcore_axis_name="s")


def _build_identity(id_ref, base):
    lanes = lax.iota(jnp.int32, 16)

    def body(i, carry):
        for k in range(SUB // 16):
            id_ref[i, pl.ds(k * 16, 16)] = base + i * SUB + k * 16 + lanes
        return carry

    lax.fori_loop(0, NSUB, body, 0)


def _load_v_into_shared(v_hbm, buf_a, shared_v, base, csem, lsem):
    def load(slot, i):
        r0 = base + i * SUB
        return pltpu.make_async_copy(v_hbm.at[pl.ds(r0, SUB)], buf_a.at[slot], csem)

    load(0, 0).start()
    load(1, 1).start()

    def body(i, carry):
        cur = lax.rem(i, NSLOT)
        pf = lax.rem(i + 2, NSLOT)
        load(cur, i).wait()
        load(pf, lax.rem(i + 2, NSUB)).start()
        ld = pltpu.make_async_copy(
            buf_a.at[cur], shared_v.at[pl.ds(base + i * SUB, SUB)], lsem)
        ld.start()
        ld.wait()
        return carry

    lax.fori_loop(0, NSUB, body, 0)
    load(NSUB % NSLOT, 0).wait()
    load((NSUB + 1) % NSLOT, 0).wait()


def _combine_into_shared(parts, buf_a, buf_b, id_ref, shared_v, base,
                         csem, lsem, ssem):
    def loads(slot, i):
        r0 = base + i * SUB
        return (
            pltpu.make_async_copy(parts.at[0, pl.ds(r0, SUB)], buf_a.at[slot], csem),
            pltpu.make_async_copy(parts.at[1, pl.ds(r0, SUB)], buf_b.at[slot], csem),
        )

    def scat(slot, i):
        return pltpu.make_async_copy(buf_b.at[slot], shared_v.at[id_ref.at[i]], ssem)

    for d in loads(0, 0) + loads(1, 1):
        d.start()

    def body(i, carry):
        cur = lax.rem(i, NSLOT)
        pf = lax.rem(i + 2, NSLOT)
        for d in loads(cur, i):
            d.wait()
        @pl.when(i >= 1)
        def _():
            scat(pf, i - 1).wait()
        for d in loads(pf, lax.rem(i + 2, NSUB)):
            d.start()
        ld = pltpu.make_async_copy(
            buf_a.at[cur], shared_v.at[pl.ds(base + i * SUB, SUB)], lsem)
        ld.start()
        ld.wait()
        scat(cur, i).start(add=True)
        return carry

    lax.fori_loop(0, NSUB, body, 0)

    scat((NSUB - 1) % NSLOT, 0).wait()
    for slot in (NSUB % NSLOT, (NSUB + 1) % NSLOT):
        for d in loads(slot, 0):
            d.wait()


def _zero_shared_out(zeros_hbm, buf, shared_out, base, zsem):
    pltpu.sync_copy(zeros_hbm, buf)

    def issue(i, carry):
        pltpu.make_async_copy(
            buf, shared_out.at[pl.ds(base + i * SUB, SUB)], zsem).start()
        return carry

    def drain(i, carry):
        pltpu.make_async_copy(
            buf, shared_out.at[pl.ds(base, SUB)], zsem).wait()
        return carry

    lax.fori_loop(0, NSUB, issue, 0)
    lax.fori_loop(0, NSUB, drain, 0)


def _edge_phase(edges_hbm, ibuf, rows, isem, gsem, ssem,
                shared_v, shared_out, w):
    big = w < NBIG
    g0 = jnp.where(big, w * 196, NBIG * 196 + (w - NBIG) * 195)
    nck = jnp.where(big, 196, 195)

    def idx_copies(slot, row):
        return (
            pltpu.make_async_copy(edges_hbm.at[1, row], ibuf.at[slot, 0], isem),
            pltpu.make_async_copy(edges_hbm.at[0, row], ibuf.at[slot, 1], isem),
        )

    def scatter_copy(slot):
        return pltpu.make_async_copy(
            rows.at[slot], shared_out.at[ibuf.at[slot, 1]], ssem)

    for d in idx_copies(0, g0):
        d.start()

    def chunk(g, carry):
        c = lax.rem(g, NSLOT)
        c1 = lax.rem(g + 1, NSLOT)
        for d in idx_copies(c, g0 + g):
            d.wait()
        @pl.when(g >= 2)
        def _():
            scatter_copy(c1).wait()
        row_pf = jnp.where(g + 1 < nck, g0 + g + 1, g0)
        for d in idx_copies(c1, row_pf):
            d.start()
        gd = pltpu.make_async_copy(shared_v.at[ibuf.at[c, 0]], rows.at[c], gsem)
        gd.start()
        gd.wait()
        scatter_copy(c).start(add=True)
        return carry

    lax.fori_loop(0, nck, chunk, 0)

    scatter_copy(0).wait()
    scatter_copy(0).wait()
    for d in idx_copies(0, g0):
        d.wait()


def _writeout_parts(parts_out, buf, shared_out, base, c, wsem):
    def wo(slot, i):
        r0 = base + i * SUB
        return pltpu.make_async_copy(
            buf.at[slot], parts_out.at[c, pl.ds(r0, SUB)], wsem)

    def body(i, carry):
        slot = lax.rem(i, NSLOT)
        @pl.when(i >= NSLOT)
        def _():
            wo(slot, 0).wait()
        pltpu.sync_copy(shared_out.at[pl.ds(base + i * SUB, SUB)], buf.at[slot])
        wo(slot, i).start()
        return carry

    lax.fori_loop(0, NSUB, body, 0)
    for _ in range(NSLOT):
        wo(0, 0).wait()


_EDGE_SCRATCH = [
    pltpu.VMEM((NSLOT, 2, CH_E), jnp.int32),
    pltpu.VMEM((NSLOT, CH_E, D), jnp.float32),
    pltpu.SemaphoreType.DMA,
    pltpu.SemaphoreType.DMA,
    pltpu.SemaphoreType.DMA,
]


@functools.partial(
    pl.kernel,
    out_type=jax.ShapeDtypeStruct((NC, NP, D), jnp.float32),
    mesh=_mesh,
    compiler_params=pltpu.CompilerParams(use_tc_tiling_on_sc=False),
    scratch_types=[
        pltpu.VMEM_SHARED((NP, D), jnp.float32),
        pltpu.VMEM_SHARED((NP, D), jnp.float32),
        pltpu.VMEM((NSLOT, SUB, D), jnp.float32),
    ] + _EDGE_SCRATCH,
)
def _step_first(v_hbm, edges_hbm, zeros_hbm, parts_out,
                shared_v, shared_out, buf_a, ibuf, rows, isem, gsem, ssem):
    c = lax.axis_index("c")
    s = lax.axis_index("s")
    base = s * ROWS_PER_TILE
    _load_v_into_shared(v_hbm, buf_a, shared_v, base, isem, gsem)
    _zero_shared_out(zeros_hbm, buf_a.at[0], shared_out, base, ssem)
    plsc.subcore_barrier()
    _edge_phase(edges_hbm, ibuf, rows, isem, gsem, ssem,
                shared_v, shared_out, c * NS + s)
    plsc.subcore_barrier()
    _writeout_parts(parts_out, buf_a, shared_out, base, c, gsem)


@functools.partial(
    pl.kernel,
    out_type=jax.ShapeDtypeStruct((NC, NP, D), jnp.float32),
    mesh=_mesh,
    compiler_params=pltpu.CompilerParams(use_tc_tiling_on_sc=False),
    scratch_types=[
        pltpu.VMEM_SHARED((NP, D), jnp.float32),
        pltpu.VMEM_SHARED((NP, D), jnp.float32),
        pltpu.VMEM((NSLOT, SUB, D), jnp.float32),
        pltpu.VMEM((NSLOT, SUB, D), jnp.float32),
        pltpu.VMEM((NSUB, SUB), jnp.int32),
    ] + _EDGE_SCRATCH,
)
def _step_mid(parts_in, edges_hbm, zeros_hbm, parts_out,
              shared_v, shared_out, buf_a, buf_b, id_ref, ibuf, rows,
              isem, gsem, ssem):
    c = lax.axis_index("c")
    s = lax.axis_index("s")
    base = s * ROWS_PER_TILE
    _build_identity(id_ref, base)
    _combine_into_shared(parts_in, buf_a, buf_b, id_ref, shared_v, base,
                         isem, gsem, ssem)
    _zero_shared_out(zeros_hbm, buf_a.at[0], shared_out, base, ssem)
    plsc.subcore_barrier()
    _edge_phase(edges_hbm, ibuf, rows, isem, gsem, ssem,
                shared_v, shared_out, c * NS + s)
    plsc.subcore_barrier()
    _writeout_parts(parts_out, buf_a, shared_out, base, c, gsem)


@functools.partial(
    pl.kernel,
    out_type=jax.ShapeDtypeStruct((NP, D), jnp.float32),
    mesh=_mesh,
    compiler_params=pltpu.CompilerParams(use_tc_tiling_on_sc=False),
    scratch_types=[
        pltpu.VMEM_SHARED((NP, D), jnp.float32),
        pltpu.VMEM((NSLOT, SUB, D), jnp.float32),
        pltpu.VMEM((NSLOT, SUB, D), jnp.float32),
        pltpu.VMEM((NSUB, SUB), jnp.int32),
        pltpu.VMEM((224, D), jnp.float32),
        pltpu.SemaphoreType.DMA,
        pltpu.SemaphoreType.DMA,
        pltpu.SemaphoreType.DMA,
    ],
)
def _step_last(parts_in, v_out, shared_v, buf_a, buf_b, id_ref, wbuf,
               csem, lsem, ssem):
    c = lax.axis_index("c")
    s = lax.axis_index("s")
    base = s * ROWS_PER_TILE
    _build_identity(id_ref, base)
    _combine_into_shared(parts_in, buf_a, buf_b, id_ref, shared_v, base,
                         csem, lsem, ssem)
    half = ROWS_PER_TILE // NC
    h0 = base + c * half
    for i in range(half // 224):
        r0 = h0 + i * 224
        pltpu.sync_copy(shared_v.at[pl.ds(r0, 224)], wbuf)
        pltpu.sync_copy(wbuf, v_out.at[pl.ds(r0, 224)])


def kernel(v0, edge_index):
    edges = edge_index.astype(jnp.int32).reshape(2, NCK, CH_E)
    v_pad = jnp.pad(v0, ((0, NP - N), (0, 0)))
    zeros = jnp.zeros((SUB, D), jnp.float32)

    parts = _step_first(v_pad, edges, zeros)
    parts = _step_mid(parts, edges, zeros)
    parts = _step_mid(parts, edges, zeros)
    v = _step_last(parts)
    return v[:N]


# speedup vs baseline: 93.0126x; 1.0203x over previous
import functools

import jax
import jax.numpy as jnp
from jax import lax
from jax.experimental import pallas as pl
from jax.experimental.pallas import tpu as pltpu
from jax.experimental.pallas import tpu_sc as plsc

N = 100000
D = 8
NC = 2
NS = 16
NW = NC * NS
ROWS_PER_TILE = 6272
NP = NS * ROWS_PER_TILE
SUB = 128
NSUB = ROWS_PER_TILE // SUB
E = 3_200_000
CH_E = 512
NCK = E // CH_E
NBIG = NCK - 195 * NW
NSLOT = 3

_mesh = plsc.VectorSubcoreMesh(core_axis_name="c", subcore_axis_name="s")


def _build_identity(id_ref, base):
    lanes = lax.iota(jnp.int32, 16)

    def body(i, carry):
        for k in range(SUB // 16):
            id_ref[i, pl.ds(k * 16, 16)] = base + i * SUB + k * 16 + lanes
        return carry

    lax.fori_loop(0, NSUB, body, 0)


def _load_v_into_shared(v_hbm, buf_a, shared_v, base, csem, lsem):
    def load(slot, i):
        r0 = base + i * SUB
        return pltpu.make_async_copy(v_hbm.at[pl.ds(r0, SUB)], buf_a.at[slot], csem)

    load(0, 0).start()
    load(1, 1).start()

    def body(i, carry):
        cur = lax.rem(i, NSLOT)
        pf = lax.rem(i + 2, NSLOT)
        load(cur, i).wait()
        load(pf, lax.rem(i + 2, NSUB)).start()
        ld = pltpu.make_async_copy(
            buf_a.at[cur], shared_v.at[pl.ds(base + i * SUB, SUB)], lsem)
        ld.start()
        ld.wait()
        return carry

    lax.fori_loop(0, NSUB, body, 0)
    load(NSUB % NSLOT, 0).wait()
    load((NSUB + 1) % NSLOT, 0).wait()


def _combine_into_shared(parts, buf_a, buf_b, id_ref, shared_v, base,
                         csem, lsem, ssem):
    def loads(slot, i):
        r0 = base + i * SUB
        return (
            pltpu.make_async_copy(parts.at[0, pl.ds(r0, SUB)], buf_a.at[slot], csem),
            pltpu.make_async_copy(parts.at[1, pl.ds(r0, SUB)], buf_b.at[slot], csem),
        )

    def scat(slot, i):
        return pltpu.make_async_copy(buf_b.at[slot], shared_v.at[id_ref.at[i]], ssem)

    for d in loads(0, 0) + loads(1, 1):
        d.start()

    def body(i, carry):
        cur = lax.rem(i, NSLOT)
        pf = lax.rem(i + 2, NSLOT)
        for d in loads(cur, i):
            d.wait()
        @pl.when(i >= 1)
        def _():
            scat(pf, i - 1).wait()
        for d in loads(pf, lax.rem(i + 2, NSUB)):
            d.start()
        ld = pltpu.make_async_copy(
            buf_a.at[cur], shared_v.at[pl.ds(base + i * SUB, SUB)], lsem)
        ld.start()
        ld.wait()
        scat(cur, i).start(add=True)
        return carry

    lax.fori_loop(0, NSUB, body, 0)

    scat((NSUB - 1) % NSLOT, 0).wait()
    for slot in (NSUB % NSLOT, (NSUB + 1) % NSLOT):
        for d in loads(slot, 0):
            d.wait()


def _zero_shared_out(zeros_hbm, buf, shared_out, base, zsem):
    pltpu.sync_copy(zeros_hbm, buf)

    def issue(i, carry):
        pltpu.make_async_copy(
            buf, shared_out.at[pl.ds(base + i * SUB, SUB)], zsem).start()
        return carry

    def drain(i, carry):
        pltpu.make_async_copy(
            buf, shared_out.at[pl.ds(base, SUB)], zsem).wait()
        return carry

    lax.fori_loop(0, NSUB, issue, 0)
    lax.fori_loop(0, NSUB, drain, 0)


def _edge_phase(edges_hbm, ibuf, rows, isem, gsem, ssem,
                shared_v, shared_out, w):
    big = w < NBIG
    g0 = jnp.where(big, w * 196, NBIG * 196 + (w - NBIG) * 195)
    nck = jnp.where(big, 196, 195)

    def idx_copies(slot, row):
        return (
            pltpu.make_async_copy(edges_hbm.at[1, row], ibuf.at[slot, 0], isem),
            pltpu.make_async_copy(edges_hbm.at[0, row], ibuf.at[slot, 1], isem),
        )

    def scatter_copy(slot):
        return pltpu.make_async_copy(
            rows.at[slot], shared_out.at[ibuf.at[slot, 1]], ssem)

    for d in idx_copies(0, g0):
        d.start()

    def chunk(g, carry):
        c = lax.rem(g, NSLOT)
        c1 = lax.rem(g + 1, NSLOT)
        for d in idx_copies(c, g0 + g):
            d.wait()
        @pl.when(g >= 2)
        def _():
            scatter_copy(c1).wait()
        row_pf = jnp.where(g + 1 < nck, g0 + g + 1, g0)
        for d in idx_copies(c1, row_pf):
            d.start()
        gd = pltpu.make_async_copy(shared_v.at[ibuf.at[c, 0]], rows.at[c], gsem)
        gd.start()
        gd.wait()
        scatter_copy(c).start(add=True)
        return carry

    lax.fori_loop(0, nck, chunk, 0)

    scatter_copy(0).wait()
    scatter_copy(0).wait()
    for d in idx_copies(0, g0):
        d.wait()


def _writeout_parts(parts_out, buf, shared_out, base, c, wsem):
    def wo(slot, i):
        r0 = base + i * SUB
        return pltpu.make_async_copy(
            buf.at[slot], parts_out.at[c, pl.ds(r0, SUB)], wsem)

    def body(i, carry):
        slot = lax.rem(i, NSLOT)
        @pl.when(i >= NSLOT)
        def _():
            wo(slot, 0).wait()
        pltpu.sync_copy(shared_out.at[pl.ds(base + i * SUB, SUB)], buf.at[slot])
        wo(slot, i).start()
        return carry

    lax.fori_loop(0, NSUB, body, 0)
    for _ in range(NSLOT):
        wo(0, 0).wait()


@functools.partial(
    pl.kernel,
    out_type=(
        jax.ShapeDtypeStruct((NP, D), jnp.float32),
        jax.ShapeDtypeStruct((2, NC, NP, D), jnp.float32),
    ),
    mesh=_mesh,
    compiler_params=pltpu.CompilerParams(use_tc_tiling_on_sc=False),
    scratch_types=[
        pltpu.VMEM_SHARED((NP, D), jnp.float32),
        pltpu.VMEM_SHARED((NP, D), jnp.float32),
        pltpu.VMEM((NSLOT, SUB, D), jnp.float32),
        pltpu.VMEM((NSLOT, SUB, D), jnp.float32),
        pltpu.VMEM((NSUB, SUB), jnp.int32),
        pltpu.VMEM((NSLOT, 2, CH_E), jnp.int32),
        pltpu.VMEM((NSLOT, CH_E, D), jnp.float32),
        pltpu.VMEM((224, D), jnp.float32),
        pltpu.SemaphoreType.DMA,
        pltpu.SemaphoreType.DMA,
        pltpu.SemaphoreType.DMA,
        pltpu.SemaphoreType.REGULAR,
    ],
)
def _fused(v_hbm, edges_hbm, zeros_hbm, v_out, parts,
           shared_v, shared_out, buf_a, buf_b, id_ref, ibuf, rows, wbuf,
           isem, gsem, ssem, xsem):
    c = lax.axis_index("c")
    s = lax.axis_index("s")
    base = s * ROWS_PER_TILE
    w = c * NS + s

    def global_barrier():
        plsc.subcore_barrier()
        pl.semaphore_signal(xsem, 1, core_index=1 - c)
        pl.semaphore_wait(xsem, 1)

    _build_identity(id_ref, base)

    _load_v_into_shared(v_hbm, buf_a, shared_v, base, isem, gsem)
    _zero_shared_out(zeros_hbm, buf_a.at[0], shared_out, base, ssem)
    plsc.subcore_barrier()
    _edge_phase(edges_hbm, ibuf, rows, isem, gsem, ssem, shared_v, shared_out, w)
    plsc.subcore_barrier()
    _writeout_parts(parts.at[0], buf_a, shared_out, base, c, gsem)

    for it in (1, 2):
        global_barrier()
        _combine_into_shared(parts.at[(it - 1) % 2], buf_a, buf_b, id_ref,
                             shared_v, base, isem, gsem, ssem)
        _zero_shared_out(zeros_hbm, buf_a.at[0], shared_out, base, ssem)
        plsc.subcore_barrier()
        _edge_phase(edges_hbm, ibuf, rows, isem, gsem, ssem,
                    shared_v, shared_out, w)
        plsc.subcore_barrier()
        _writeout_parts(parts.at[it % 2], buf_a, shared_out, base, c, gsem)

    global_barrier()
    _combine_into_shared(parts.at[0], buf_a, buf_b, id_ref, shared_v, base,
                         isem, gsem, ssem)
    half = ROWS_PER_TILE // NC
    h0 = base + c * half
    for i in range(half // 224):
        r0 = h0 + i * 224
        pltpu.sync_copy(shared_v.at[pl.ds(r0, 224)], wbuf)
        pltpu.sync_copy(wbuf, v_out.at[pl.ds(r0, 224)])


def kernel(v0, edge_index):
    edges = edge_index.astype(jnp.int32).reshape(2, NCK, CH_E)
    v_pad = jnp.pad(v0, ((0, NP - N), (0, 0)))
    zeros = jnp.zeros((SUB, D), jnp.float32)

    v, _ = _fused(v_pad, edges, zeros)
    return v[:N]


# speedup vs baseline: 102.4957x; 1.1020x over previous
import functools

import jax
import jax.numpy as jnp
from jax import lax
from jax.experimental import pallas as pl
from jax.experimental.pallas import tpu as pltpu
from jax.experimental.pallas import tpu_sc as plsc

N = 100000
D = 8
NC = 2
NS = 16
NW = NC * NS
ROWS_PER_TILE = 6272
NP = NS * ROWS_PER_TILE
SUB = 128
NSUB = ROWS_PER_TILE // SUB
LAST_NSUB = 46
LAST_TAIL = N - 15 * ROWS_PER_TILE - LAST_NSUB * SUB
E = 3_200_000
CH_E = 512
NCK = E // CH_E
NBIG = NCK - 195 * NW
NSLOT = 3
WOUT = 125

_mesh = plsc.VectorSubcoreMesh(core_axis_name="c", subcore_axis_name="s")


def _build_identity(id_ref, base):
    lanes = lax.iota(jnp.int32, 16)

    def body(i, carry):
        for k in range(SUB // 16):
            id_ref[i, pl.ds(k * 16, 16)] = base + i * SUB + k * 16 + lanes
        return carry

    lax.fori_loop(0, NSUB, body, 0)


def _load_v_into_shared(v_hbm, buf_a, shared_v, base, s, csem, lsem):
    nsub = jnp.where(s == NS - 1, LAST_NSUB, NSUB)

    def load(slot, i):
        r0 = base + i * SUB
        return pltpu.make_async_copy(v_hbm.at[pl.ds(r0, SUB)], buf_a.at[slot], csem)

    load(0, 0).start()
    load(1, 1).start()

    def body(i, carry):
        cur = lax.rem(i, NSLOT)
        pf = lax.rem(i + 2, NSLOT)
        load(cur, i).wait()
        load(pf, lax.rem(i + 2, nsub)).start()
        ld = pltpu.make_async_copy(
            buf_a.at[cur], shared_v.at[pl.ds(base + i * SUB, SUB)], lsem)
        ld.start()
        ld.wait()
        return carry

    lax.fori_loop(0, nsub, body, 0)
    load(0, 0).wait()
    load(0, 0).wait()

    @pl.when(s == NS - 1)
    def _():
        r0 = base + LAST_NSUB * SUB
        t = pltpu.make_async_copy(
            v_hbm.at[pl.ds(r0, LAST_TAIL)], buf_a.at[0, pl.ds(0, LAST_TAIL)], csem)
        t.start()
        t.wait()
        t2 = pltpu.make_async_copy(
            buf_a.at[0, pl.ds(0, LAST_TAIL)], shared_v.at[pl.ds(r0, LAST_TAIL)], lsem)
        t2.start()
        t2.wait()


def _combine_into_shared(parts, buf_a, buf_b, id_ref, shared_v, base,
                         csem, lsem, ssem):
    def loads(slot, i):
        r0 = base + i * SUB
        return (
            pltpu.make_async_copy(parts.at[0, pl.ds(r0, SUB)], buf_a.at[slot], csem),
            pltpu.make_async_copy(parts.at[1, pl.ds(r0, SUB)], buf_b.at[slot], csem),
        )

    def scat(slot, i):
        return pltpu.make_async_copy(buf_b.at[slot], shared_v.at[id_ref.at[i]], ssem)

    for d in loads(0, 0) + loads(1, 1):
        d.start()

    def body(i, carry):
        cur = lax.rem(i, NSLOT)
        pf = lax.rem(i + 2, NSLOT)
        for d in loads(cur, i):
            d.wait()
        @pl.when(i >= 1)
        def _():
            scat(pf, i - 1).wait()
        for d in loads(pf, lax.rem(i + 2, NSUB)):
            d.start()
        ld = pltpu.make_async_copy(
            buf_a.at[cur], shared_v.at[pl.ds(base + i * SUB, SUB)], lsem)
        ld.start()
        ld.wait()
        scat(cur, i).start(add=True)
        return carry

    lax.fori_loop(0, NSUB, body, 0)

    scat((NSUB - 1) % NSLOT, 0).wait()
    for slot in (NSUB % NSLOT, (NSUB + 1) % NSLOT):
        for d in loads(slot, 0):
            d.wait()


def _zero_shared_out(zeros_hbm, buf, shared_out, base, zsem):
    pltpu.sync_copy(zeros_hbm, buf)

    def issue(i, carry):
        pltpu.make_async_copy(
            buf, shared_out.at[pl.ds(base + i * SUB, SUB)], zsem).start()
        return carry

    def drain(i, carry):
        pltpu.make_async_copy(
            buf, shared_out.at[pl.ds(base, SUB)], zsem).wait()
        return carry

    lax.fori_loop(0, NSUB, issue, 0)
    lax.fori_loop(0, NSUB, drain, 0)


def _edge_phase(edges_hbm, ibuf, rows, isem, gsem, ssem,
                shared_v, shared_out, w):
    big = w < NBIG
    g0 = jnp.where(big, w * 196, NBIG * 196 + (w - NBIG) * 195)
    nck = jnp.where(big, 196, 195)

    def idx_copies(slot, row):
        e0 = row * CH_E
        return (
            pltpu.make_async_copy(
                edges_hbm.at[1, pl.ds(e0, CH_E)], ibuf.at[slot, 0], isem),
            pltpu.make_async_copy(
                edges_hbm.at[0, pl.ds(e0, CH_E)], ibuf.at[slot, 1], isem),
        )

    def scatter_copy(slot):
        return pltpu.make_async_copy(
            rows.at[slot], shared_out.at[ibuf.at[slot, 1]], ssem)

    for d in idx_copies(0, g0):
        d.start()

    def chunk(g, carry):
        c = lax.rem(g, NSLOT)
        c1 = lax.rem(g + 1, NSLOT)
        for d in idx_copies(c, g0 + g):
            d.wait()
        @pl.when(g >= 2)
        def _():
            scatter_copy(c1).wait()
        row_pf = jnp.where(g + 1 < nck, g0 + g + 1, g0)
        for d in idx_copies(c1, row_pf):
            d.start()
        gd = pltpu.make_async_copy(shared_v.at[ibuf.at[c, 0]], rows.at[c], gsem)
        gd.start()
        gd.wait()
        scatter_copy(c).start(add=True)
        return carry

    lax.fori_loop(0, nck, chunk, 0)

    scatter_copy(0).wait()
    scatter_copy(0).wait()
    for d in idx_copies(0, g0):
        d.wait()


def _writeout_parts(parts_out, buf, shared_out, base, c, wsem):
    def wo(slot, i):
        r0 = base + i * SUB
        return pltpu.make_async_copy(
            buf.at[slot], parts_out.at[c, pl.ds(r0, SUB)], wsem)

    def body(i, carry):
        slot = lax.rem(i, NSLOT)
        @pl.when(i >= NSLOT)
        def _():
            wo(slot, 0).wait()
        pltpu.sync_copy(shared_out.at[pl.ds(base + i * SUB, SUB)], buf.at[slot])
        wo(slot, i).start()
        return carry

    lax.fori_loop(0, NSUB, body, 0)
    for _ in range(NSLOT):
        wo(0, 0).wait()


@functools.partial(
    pl.kernel,
    out_type=jax.ShapeDtypeStruct((N, D), jnp.float32),
    mesh=_mesh,
    compiler_params=pltpu.CompilerParams(use_tc_tiling_on_sc=False),
    scratch_types=[
        pltpu.HBM((2, NC, NP, D), jnp.float32),
        pltpu.VMEM_SHARED((NP, D), jnp.float32),
        pltpu.VMEM_SHARED((NP, D), jnp.float32),
        pltpu.VMEM((NSLOT, SUB, D), jnp.float32),
        pltpu.VMEM((NSLOT, SUB, D), jnp.float32),
        pltpu.VMEM((NSUB, SUB), jnp.int32),
        pltpu.VMEM((NSLOT, 2, CH_E), jnp.int32),
        pltpu.VMEM((NSLOT, CH_E, D), jnp.float32),
        pltpu.VMEM((WOUT, D), jnp.float32),
        pltpu.SemaphoreType.DMA,
        pltpu.SemaphoreType.DMA,
        pltpu.SemaphoreType.DMA,
        pltpu.SemaphoreType.REGULAR,
    ],
)
def _fused(v_hbm, edges_hbm, zeros_hbm, v_out, parts,
           shared_v, shared_out, buf_a, buf_b, id_ref, ibuf, rows, wbuf,
           isem, gsem, ssem, xsem):
    c = lax.axis_index("c")
    s = lax.axis_index("s")
    base = s * ROWS_PER_TILE
    w = c * NS + s

    def global_barrier():
        plsc.subcore_barrier()
        pl.semaphore_signal(xsem, 1, core_index=1 - c)
        pl.semaphore_wait(xsem, 1)

    _build_identity(id_ref, base)

    _load_v_into_shared(v_hbm, buf_a, shared_v, base, s, isem, gsem)
    _zero_shared_out(zeros_hbm, buf_a.at[0], shared_out, base, ssem)
    plsc.subcore_barrier()
    _edge_phase(edges_hbm, ibuf, rows, isem, gsem, ssem, shared_v, shared_out, w)
    plsc.subcore_barrier()
    _writeout_parts(parts.at[0], buf_a, shared_out, base, c, gsem)

    for it in (1, 2):
        global_barrier()
        _combine_into_shared(parts.at[(it - 1) % 2], buf_a, buf_b, id_ref,
                             shared_v, base, isem, gsem, ssem)
        _zero_shared_out(zeros_hbm, buf_a.at[0], shared_out, base, ssem)
        plsc.subcore_barrier()
        _edge_phase(edges_hbm, ibuf, rows, isem, gsem, ssem,
                    shared_v, shared_out, w)
        plsc.subcore_barrier()
        _writeout_parts(parts.at[it % 2], buf_a, shared_out, base, c, gsem)

    global_barrier()
    _combine_into_shared(parts.at[0], buf_a, buf_b, id_ref, shared_v, base,
                         isem, gsem, ssem)
    plsc.subcore_barrier()
    o0 = w * (N // NW)
    for i in range(N // NW // WOUT):
        r0 = o0 + i * WOUT
        pltpu.sync_copy(shared_v.at[pl.ds(r0, WOUT)], wbuf)
        pltpu.sync_copy(wbuf, v_out.at[pl.ds(r0, WOUT)])


def kernel(v0, edge_index):
    edges = edge_index.astype(jnp.int32)
    zeros = jnp.zeros((SUB, D), jnp.float32)
    return _fused(v0, edges, zeros)
